# baseline (speedup 1.0000x reference)
import functools

import jax
import jax.numpy as jnp
from jax import lax
from jax.experimental import pallas as pl
from jax.experimental.pallas import tpu as pltpu
from jax.experimental.pallas import tpu_sc as plsc

N = 10000
D = 128
G = 128
EPS = 1e-5

NC = 2
NS = 16
NW = NC * NS
CHUNK = 128

ACC_ROWS = 10112
ROWS_PER_TILE = ACC_ROWS // NS

def _get_mesh():
  return plsc.VectorSubcoreMesh(
      core_axis_name="c", subcore_axis_name="s", num_cores=NC, num_subcores=NS)


def _dot(a, b):
  return lax.dot_general(
      a, b, (((1,), (0,)), ((), ())),
      precision=lax.Precision.DEFAULT, preferred_element_type=jnp.float32)


def _make_sc_deg(k0, k1):
  kmax = max(k0, k1)

  @functools.partial(
      pl.kernel,
      out_type=jax.ShapeDtypeStruct((NC, ACC_ROWS, D), jnp.float32),
      mesh=_get_mesh(),
      scratch_types=[
          pltpu.VMEM((kmax, CHUNK), jnp.int32),
          pltpu.VMEM((CHUNK, D), jnp.float32),
          pltpu.VMEM_SHARED((ACC_ROWS, D), jnp.float32),
      ],
  )
  def sc_deg(dst_hbm, ones_hbm, zeros_hbm, out_hbm, dst_v, ones_v, acc):
    c = lax.axis_index("c")
    s = lax.axis_index("s")
    w = s * NC + c
    row0 = s * ROWS_PER_TILE
    pltpu.sync_copy(zeros_hbm, acc.at[pl.ds(row0, ROWS_PER_TILE)])
    pltpu.sync_copy(dst_hbm.at[w], dst_v)
    pltpu.sync_copy(ones_hbm, ones_v)
    plsc.subcore_barrier()

    def body(j, carry):
      pltpu.sync_copy(ones_v, acc.at[dst_v.at[j]], add=True)
      return carry

    def run(k):
      lax.fori_loop(0, k, body, 0, unroll=False)

    if k0 == k1:
      run(k0)
    else:
      @pl.when(c == 0)
      def _():
        run(k0)

      @pl.when(c == 1)
      def _():
        run(k1)

    plsc.subcore_barrier()
    pltpu.sync_copy(acc.at[pl.ds(row0, ROWS_PER_TILE)],
                    out_hbm.at[c, pl.ds(row0, ROWS_PER_TILE)])

  return sc_deg


STAGE = 32


def _stages(k):
  out = []
  off = 0
  while off < k:
    sz = min(STAGE, k - off)
    out.append((off, sz))
    off += sz
  return out


def _make_sc_agg(k0, k1):
  @functools.partial(
      pl.kernel,
      out_type=jax.ShapeDtypeStruct((NC, ACC_ROWS, D), jnp.float32),
      mesh=_get_mesh(),
      scratch_types=[
          pltpu.VMEM((STAGE, CHUNK), jnp.int32),
          pltpu.VMEM((STAGE, CHUNK), jnp.int32),
          pltpu.VMEM((CHUNK, D), jnp.float32),
          pltpu.VMEM((CHUNK, D), jnp.float32),
          pltpu.SemaphoreType.DMA,
          pltpu.SemaphoreType.DMA,
          pltpu.VMEM_SHARED((ACC_ROWS, D), jnp.float32),
      ],
  )
  def sc_agg(ht_hbm, src_hbm, dst_hbm, zeros_hbm, out_hbm,
             src_v, dst_v, buf_a, buf_b, sem_a, sem_b, acc):
    c = lax.axis_index("c")
    s = lax.axis_index("s")
    w = s * NC + c
    row0 = s * ROWS_PER_TILE
    pltpu.sync_copy(zeros_hbm, acc.at[pl.ds(row0, ROWS_PER_TILE)])
    plsc.subcore_barrier()

    def body(j, carry):
      i0 = 2 * j
      i1 = 2 * j + 1
      ga = pltpu.make_async_copy(ht_hbm.at[src_v.at[i0]], buf_a, sem_a)
      ga.start()
      gb = pltpu.make_async_copy(ht_hbm.at[src_v.at[i1]], buf_b, sem_b)
      gb.start()
      ga.wait()
      pltpu.sync_copy(buf_a, acc.at[dst_v.at[i0]], add=True)
      gb.wait()
      pltpu.sync_copy(buf_b, acc.at[dst_v.at[i1]], add=True)
      return carry

    def run(k):
      for off, sz in _stages(k):
        pltpu.sync_copy(src_hbm.at[w, pl.ds(off, sz)], src_v.at[pl.ds(0, sz)])
        pltpu.sync_copy(dst_hbm.at[w, pl.ds(off, sz)], dst_v.at[pl.ds(0, sz)])
        lax.fori_loop(0, sz // 2, body, 0, unroll=False)

    if k0 == k1:
      run(k0)
    else:
      @pl.when(c == 0)
      def _():
        run(k0)

      @pl.when(c == 1)
      def _():
        run(k1)

    plsc.subcore_barrier()
    pltpu.sync_copy(acc.at[pl.ds(row0, ROWS_PER_TILE)],
                    out_hbm.at[c, pl.ds(row0, ROWS_PER_TILE)])

  return sc_agg


def _tc_pre_body(degp_ref, x_ref, w1_ref, dinv_ref, ht_ref):
  cnt = degp_ref[0, :N, :1] + degp_ref[1, :N, :1]
  dinv = lax.rsqrt(cnt + 1.0)
  dinv_ref[...] = dinv
  h = _dot(x_ref[...], w1_ref[...])
  ht_ref[...] = h * dinv


def _tc_mid_body(acc_ref, ht_ref, dinv_ref, b_ref, g_ref, be_ref, wn_ref,
                 out_ref):
  dinv = dinv_ref[...]
  agg = acc_ref[0, :N, :] + acc_ref[1, :N, :] + ht_ref[...]
  z = agg * dinv + b_ref[...]
  r = jnp.maximum(z, 0.0)
  mu = jnp.mean(r, axis=0, keepdims=True)
  xc = r - mu
  var = jnp.mean(xc * xc, axis=0, keepdims=True)
  hbn = xc * lax.rsqrt(var + EPS) * g_ref[...] + be_ref[...]
  out_ref[...] = _dot(hbn, wn_ref[...]) * dinv


def _tc_fin_body(acc_ref, ht_ref, dinv_ref, b_ref, g_ref, be_ref,
                 batch_ref, wl_ref, bl_ref,
                 logits_ref, node_ref, graph_ref):
  dinv = dinv_ref[...]
  agg = acc_ref[0, :N, :] + acc_ref[1, :N, :] + ht_ref[...]
  z = agg * dinv + b_ref[...]
  r = jnp.maximum(z, 0.0)
  mu = jnp.mean(r, axis=0, keepdims=True)
  xc = r - mu
  var = jnp.mean(xc * xc, axis=0, keepdims=True)
  node = xc * lax.rsqrt(var + EPS) * g_ref[...] + be_ref[...]
  node_ref[...] = node

  onehot = (batch_ref[...] == lax.broadcasted_iota(jnp.int32, (N, G), 1))
  onehot = onehot.astype(jnp.float32)
  sums = lax.dot_general(
      onehot, node, (((0,), (0,)), ((), ())),
      precision=lax.Precision.HIGHEST, preferred_element_type=jnp.float32)
  ones_col = jnp.ones((N, 1), jnp.float32)
  counts = lax.dot_general(
      onehot, ones_col, (((0,), (0,)), ((), ())),
      precision=lax.Precision.HIGHEST, preferred_element_type=jnp.float32)
  graph = sums / jnp.maximum(counts, 1.0)
  graph_ref[...] = graph
  logits_ref[...] = _dot(graph, wl_ref[...]) + bl_ref[...]


_f32 = jnp.float32

_tc_pre = pl.pallas_call(
    _tc_pre_body,
    out_shape=(jax.ShapeDtypeStruct((N, 1), _f32),
               jax.ShapeDtypeStruct((N, D), _f32)))

_tc_mid = pl.pallas_call(
    _tc_mid_body,
    out_shape=jax.ShapeDtypeStruct((N, D), _f32))

_tc_fin = pl.pallas_call(
    _tc_fin_body,
    out_shape=(jax.ShapeDtypeStruct((G, 10), _f32),
               jax.ShapeDtypeStruct((N, D), _f32),
               jax.ShapeDtypeStruct((G, G), _f32)))


_SPLIT = 4


def _pack_chunks(vals, k0, k1, pad_tail):
  kmax = max(k0, k1)
  chunks = vals.reshape(-1, CHUNK)
  c0 = chunks[:NS * k0].reshape(NS, k0, CHUNK)
  c1 = chunks[NS * k0:].reshape(NS, k1, CHUNK)
  c0 = jnp.pad(c0, ((0, 0), (0, kmax - k0), (0, 0)), constant_values=pad_tail)
  c1 = jnp.pad(c1, ((0, 0), (0, kmax - k1), (0, 0)), constant_values=pad_tail)
  return jnp.stack([c0, c1], axis=1).reshape(NW, kmax, CHUNK)


def kernel(x, edge_index, batch, W1, b1, g1, be1, W2, b2, g2, be2,
           W3, b3, g3, be3, Wl, bl):
  e = edge_index.shape[1]
  tot = -(-e // (NS * CHUNK))
  tot = -(-tot // 32) * 32
  k1 = 0
  k0 = tot - k1
  pad = NS * tot * CHUNK - e

  dump = N + (jnp.arange(pad, dtype=edge_index.dtype) % (ACC_ROWS - N))
  src = _pack_chunks(jnp.pad(edge_index[0], (0, pad)), k0, k1, 0)
  dst = _pack_chunks(jnp.concatenate([edge_index[1], dump]), k0, k1, N)

  zeros_acc = jnp.zeros((ROWS_PER_TILE, D), _f32)
  ones_blk = jnp.ones((CHUNK, D), _f32)

  sc_deg = _make_sc_deg(k0, k1)
  sc_agg = _make_sc_agg(k0, k1)

  degp = sc_deg(dst, ones_blk, zeros_acc)
  dinv, ht1 = _tc_pre(degp, x, W1)

  acc1 = sc_agg(ht1, src, dst, zeros_acc)
  ht2 = _tc_mid(acc1, ht1, dinv, b1.reshape(1, D), g1.reshape(1, D),
                be1.reshape(1, D), W2)

  acc2 = sc_agg(ht2, src, dst, zeros_acc)
  ht3 = _tc_mid(acc2, ht2, dinv, b2.reshape(1, D), g2.reshape(1, D),
                be2.reshape(1, D), W3)

  acc3 = sc_agg(ht3, src, dst, zeros_acc)
  logits, node_embed, graph_embed = _tc_fin(
      acc3, ht3, dinv, b3.reshape(1, D), g3.reshape(1, D), be3.reshape(1, D),
      batch.reshape(N, 1), Wl, bl.reshape(1, 10))

  return (logits, node_embed, graph_embed)


# speedup vs baseline: 1.5092x; 1.5092x over previous
import functools

import jax
import jax.numpy as jnp
from jax import lax
from jax.experimental import pallas as pl
from jax.experimental.pallas import tpu as pltpu
from jax.experimental.pallas import tpu_sc as plsc

N = 10000
D = 128
G = 128
EPS = 1e-5

NC = 2
NS = 16
NW = NC * NS
CHUNK = 128

ACC_ROWS = 10112
ROWS_PER_TILE = ACC_ROWS // NS

def _get_mesh():
  return plsc.VectorSubcoreMesh(
      core_axis_name="c", subcore_axis_name="s", num_cores=NC, num_subcores=NS)


def _dot(a, b):
  return lax.dot_general(
      a, b, (((1,), (0,)), ((), ())),
      precision=lax.Precision.DEFAULT, preferred_element_type=jnp.float32)


def _make_sc_deg(k0, k1):
  kmax = max(k0, k1)

  @functools.partial(
      pl.kernel,
      out_type=jax.ShapeDtypeStruct((NC, ACC_ROWS, D), jnp.float32),
      mesh=_get_mesh(),
      scratch_types=[
          pltpu.VMEM((kmax, CHUNK), jnp.int32),
          pltpu.VMEM((CHUNK, D), jnp.float32),
          pltpu.VMEM_SHARED((ACC_ROWS, D), jnp.float32),
      ],
  )
  def sc_deg(dst_hbm, ones_hbm, zeros_hbm, out_hbm, dst_v, ones_v, acc):
    c = lax.axis_index("c")
    s = lax.axis_index("s")
    w = s * NC + c
    row0 = s * ROWS_PER_TILE
    pltpu.sync_copy(zeros_hbm, acc.at[pl.ds(row0, ROWS_PER_TILE)])
    pltpu.sync_copy(dst_hbm.at[w], dst_v)
    pltpu.sync_copy(ones_hbm, ones_v)
    plsc.subcore_barrier()

    def body(j, carry):
      pltpu.sync_copy(ones_v, acc.at[dst_v.at[j]], add=True)
      return carry

    def run(k):
      lax.fori_loop(0, k, body, 0, unroll=False)

    if k0 == k1:
      run(k0)
    else:
      @pl.when(c == 0)
      def _():
        run(k0)

      @pl.when(c == 1)
      def _():
        run(k1)

    plsc.subcore_barrier()
    pltpu.sync_copy(acc.at[pl.ds(row0, ROWS_PER_TILE)],
                    out_hbm.at[c, pl.ds(row0, ROWS_PER_TILE)])

  return sc_deg


STAGE = 32


def _stages(k):
  out = []
  off = 0
  while off < k:
    sz = min(STAGE, k - off)
    out.append((off, sz))
    off += sz
  return out


def _make_sc_agg(k0, k1):
  @functools.partial(
      pl.kernel,
      out_type=jax.ShapeDtypeStruct((NC, ACC_ROWS, D), jnp.float32),
      mesh=_get_mesh(),
      scratch_types=[
          pltpu.VMEM((STAGE, CHUNK), jnp.int32),
          pltpu.VMEM((STAGE, CHUNK), jnp.int32),
          pltpu.VMEM((CHUNK, D), jnp.float32),
          pltpu.VMEM((CHUNK, D), jnp.float32),
          pltpu.SemaphoreType.DMA,
          pltpu.SemaphoreType.DMA,
          pltpu.VMEM_SHARED((ACC_ROWS, D), jnp.float32),
      ],
  )
  def sc_agg(ht_hbm, src_hbm, dst_hbm, zeros_hbm, out_hbm,
             src_v, dst_v, buf_a, buf_b, sem_a, sem_b, acc):
    c = lax.axis_index("c")
    s = lax.axis_index("s")
    w = s * NC + c
    row0 = s * ROWS_PER_TILE
    pltpu.sync_copy(zeros_hbm, acc.at[pl.ds(row0, ROWS_PER_TILE)])
    plsc.subcore_barrier()

    def body(j, carry):
      i0 = 2 * j
      i1 = 2 * j + 1
      ga = pltpu.make_async_copy(ht_hbm.at[src_v.at[i0]], buf_a, sem_a)
      ga.start()
      gb = pltpu.make_async_copy(ht_hbm.at[src_v.at[i1]], buf_b, sem_b)
      gb.start()
      ga.wait()
      pltpu.sync_copy(buf_a, acc.at[dst_v.at[i0]], add=True)
      gb.wait()
      pltpu.sync_copy(buf_b, acc.at[dst_v.at[i1]], add=True)
      return carry

    def run(k):
      for off, sz in _stages(k):
        pltpu.sync_copy(src_hbm.at[w, pl.ds(off, sz)], src_v.at[pl.ds(0, sz)])
        pltpu.sync_copy(dst_hbm.at[w, pl.ds(off, sz)], dst_v.at[pl.ds(0, sz)])
        lax.fori_loop(0, sz // 2, body, 0, unroll=False)

    if k0 == k1:
      run(k0)
    else:
      @pl.when(c == 0)
      def _():
        run(k0)

      @pl.when(c == 1)
      def _():
        run(k1)

    plsc.subcore_barrier()
    pltpu.sync_copy(acc.at[pl.ds(row0, ROWS_PER_TILE)],
                    out_hbm.at[c, pl.ds(row0, ROWS_PER_TILE)])

  return sc_agg


def _tc_pre_body(degp_ref, x_ref, w1_ref, dinv_ref, ht_ref):
  cnt = degp_ref[0, :N, :1] + degp_ref[1, :N, :1]
  dinv = lax.rsqrt(cnt + 1.0)
  dinv_ref[...] = dinv
  h = _dot(x_ref[...], w1_ref[...])
  ht_ref[...] = h * dinv


def _tc_mid_body(acc_ref, ht_ref, dinv_ref, b_ref, g_ref, be_ref, wn_ref,
                 out_ref):
  dinv = dinv_ref[...]
  agg = acc_ref[0, :N, :] + acc_ref[1, :N, :] + ht_ref[...]
  z = agg * dinv + b_ref[...]
  r = jnp.maximum(z, 0.0)
  mu = jnp.mean(r, axis=0, keepdims=True)
  xc = r - mu
  var = jnp.mean(xc * xc, axis=0, keepdims=True)
  hbn = xc * lax.rsqrt(var + EPS) * g_ref[...] + be_ref[...]
  out_ref[...] = _dot(hbn, wn_ref[...]) * dinv


def _tc_fin_body(acc_ref, ht_ref, dinv_ref, b_ref, g_ref, be_ref,
                 batch_ref, wl_ref, bl_ref,
                 logits_ref, node_ref, graph_ref):
  dinv = dinv_ref[...]
  agg = acc_ref[0, :N, :] + acc_ref[1, :N, :] + ht_ref[...]
  z = agg * dinv + b_ref[...]
  r = jnp.maximum(z, 0.0)
  mu = jnp.mean(r, axis=0, keepdims=True)
  xc = r - mu
  var = jnp.mean(xc * xc, axis=0, keepdims=True)
  node = xc * lax.rsqrt(var + EPS) * g_ref[...] + be_ref[...]
  node_ref[...] = node

  onehot = (batch_ref[...] == lax.broadcasted_iota(jnp.int32, (N, G), 1))
  onehot = onehot.astype(jnp.float32)
  sums = lax.dot_general(
      onehot, node, (((0,), (0,)), ((), ())),
      precision=lax.Precision.HIGHEST, preferred_element_type=jnp.float32)
  ones_col = jnp.ones((N, 1), jnp.float32)
  counts = lax.dot_general(
      onehot, ones_col, (((0,), (0,)), ((), ())),
      precision=lax.Precision.HIGHEST, preferred_element_type=jnp.float32)
  graph = sums / jnp.maximum(counts, 1.0)
  graph_ref[...] = graph
  logits_ref[...] = _dot(graph, wl_ref[...]) + bl_ref[...]


_f32 = jnp.float32

_tc_pre = pl.pallas_call(
    _tc_pre_body,
    out_shape=(jax.ShapeDtypeStruct((N, 1), _f32),
               jax.ShapeDtypeStruct((N, D), _f32)))

_tc_mid = pl.pallas_call(
    _tc_mid_body,
    out_shape=jax.ShapeDtypeStruct((N, D), _f32))

_tc_fin = pl.pallas_call(
    _tc_fin_body,
    out_shape=(jax.ShapeDtypeStruct((G, 10), _f32),
               jax.ShapeDtypeStruct((N, D), _f32),
               jax.ShapeDtypeStruct((G, G), _f32)))


def _pack_chunks(vals, k0, k1, pad_tail):
  kmax = max(k0, k1)
  chunks = vals.reshape(-1, CHUNK)
  c0 = chunks[:NS * k0].reshape(NS, k0, CHUNK)
  c1 = chunks[NS * k0:].reshape(NS, k1, CHUNK)
  c0 = jnp.pad(c0, ((0, 0), (0, kmax - k0), (0, 0)), constant_values=pad_tail)
  c1 = jnp.pad(c1, ((0, 0), (0, kmax - k1), (0, 0)), constant_values=pad_tail)
  return jnp.stack([c0, c1], axis=1).reshape(NW, kmax, CHUNK)


def kernel(x, edge_index, batch, W1, b1, g1, be1, W2, b2, g2, be2,
           W3, b3, g3, be3, Wl, bl):
  e = edge_index.shape[1]
  tot = -(-e // (NS * CHUNK))
  tot = -(-tot // 32) * 32
  k1 = max(8, (tot // 10 // 8) * 8)
  k0 = tot - k1
  pad = NS * tot * CHUNK - e

  dump = N + (jnp.arange(pad, dtype=edge_index.dtype) % (ACC_ROWS - N))
  src = _pack_chunks(jnp.pad(edge_index[0], (0, pad)), k0, k1, 0)
  dst = _pack_chunks(jnp.concatenate([edge_index[1], dump]), k0, k1, N)

  zeros_acc = jnp.zeros((ROWS_PER_TILE, D), _f32)
  ones_blk = jnp.ones((CHUNK, D), _f32)

  sc_deg = _make_sc_deg(k0, k1)
  sc_agg = _make_sc_agg(k0, k1)

  degp = sc_deg(dst, ones_blk, zeros_acc)
  dinv, ht1 = _tc_pre(degp, x, W1)

  acc1 = sc_agg(ht1, src, dst, zeros_acc)
  ht2 = _tc_mid(acc1, ht1, dinv, b1.reshape(1, D), g1.reshape(1, D),
                be1.reshape(1, D), W2)

  acc2 = sc_agg(ht2, src, dst, zeros_acc)
  ht3 = _tc_mid(acc2, ht2, dinv, b2.reshape(1, D), g2.reshape(1, D),
                be2.reshape(1, D), W3)

  acc3 = sc_agg(ht3, src, dst, zeros_acc)
  logits, node_embed, graph_embed = _tc_fin(
      acc3, ht3, dinv, b3.reshape(1, D), g3.reshape(1, D), be3.reshape(1, D),
      batch.reshape(N, 1), Wl, bl.reshape(1, 10))

  return (logits, node_embed, graph_embed)
